# Initial kernel scaffold; baseline (speedup 1.0000x reference)
import jax
import jax.numpy as jnp
from jax.experimental import pallas as pl


def kernel(X, adj_t, tuples_coo):
    raise NotImplementedError("write your pallas kernel here")


# speedup vs baseline: 5.3520x; 5.3520x over previous
import functools

import jax
import jax.numpy as jnp
from jax import lax
from jax.experimental import pallas as pl
from jax.experimental.pallas import tpu as pltpu
from jax.experimental.pallas import tpu_sc as plsc

V = 10000
D = 128
B = 320000
L = 16
NC = 2
NS = 16
NW = NC * NS
BPW = B // NW
C = 80
NCHUNK = BPW // C

_mesh = plsc.VectorSubcoreMesh(core_axis_name="c", subcore_axis_name="s")


@functools.partial(
    pl.kernel,
    mesh=_mesh,
    out_type=jax.ShapeDtypeStruct((B, D), jnp.float32),
    scratch_types=[
        pltpu.VMEM((C,), jnp.int32),
        pltpu.VMEM((C,), jnp.int32),
        pltpu.VMEM((C, D), jnp.float32),
        pltpu.VMEM((C, D), jnp.float32),
        pltpu.SemaphoreType.DMA,
        pltpu.SemaphoreType.DMA,
    ],
)
def _product_tuple(x_hbm, idx0_hbm, idx1_hbm, out_hbm,
                   idx0_v, idx1_v, rows0_v, rows1_v, sem0, sem1):
    wid = lax.axis_index("s") * NC + lax.axis_index("c")
    base = wid * BPW

    def chunk_body(c, carry):
        off = base + c * C
        pltpu.sync_copy(idx0_hbm.at[pl.ds(off, C)], idx0_v)
        pltpu.sync_copy(idx1_hbm.at[pl.ds(off, C)], idx1_v)
        cp0 = pltpu.async_copy(x_hbm.at[idx0_v], rows0_v, sem0)
        cp1 = pltpu.async_copy(x_hbm.at[idx1_v], rows1_v, sem1)
        cp0.wait()
        cp1.wait()

        def row_body(r, rcarry):
            for j in range(D // L):
                s = pl.ds(j * L, L)
                rows0_v[r, s] = rows0_v[r, s] * rows1_v[r, s]
            return rcarry

        lax.fori_loop(0, C, row_body, 0)
        pltpu.sync_copy(rows0_v, out_hbm.at[pl.ds(off, C)])
        return carry

    lax.fori_loop(0, NCHUNK, chunk_body, 0)


def kernel(X, adj_t, tuples_coo):
    del adj_t
    idx0 = tuples_coo[0]
    idx1 = tuples_coo[1]
    return _product_tuple(X, idx0, idx1)


# speedup vs baseline: 7.3112x; 1.3661x over previous
import functools

import jax
import jax.numpy as jnp
from jax import lax
from jax.experimental import pallas as pl
from jax.experimental.pallas import tpu as pltpu
from jax.experimental.pallas import tpu_sc as plsc

V = 10000
D = 128
B = 320000
L = 16
NC = 2
NS = 16
NW = NC * NS
BPW = B // NW
C = 40
N = BPW // C

_mesh = plsc.VectorSubcoreMesh(core_axis_name="c", subcore_axis_name="s")


@functools.partial(
    pl.kernel,
    mesh=_mesh,
    out_type=jax.ShapeDtypeStruct((B, D), jnp.float32),
    scratch_types=[
        pltpu.VMEM((C,), jnp.int32),
        pltpu.VMEM((C,), jnp.int32),
        pltpu.VMEM((C,), jnp.int32),
        pltpu.VMEM((C,), jnp.int32),
        pltpu.VMEM((C, D), jnp.float32),
        pltpu.VMEM((C, D), jnp.float32),
        pltpu.VMEM((C, D), jnp.float32),
        pltpu.VMEM((C, D), jnp.float32),
        pltpu.VMEM((C, D), jnp.float32),
        pltpu.VMEM((C, D), jnp.float32),
        pltpu.SemaphoreType.DMA,
        pltpu.SemaphoreType.DMA,
        pltpu.SemaphoreType.DMA,
        pltpu.SemaphoreType.DMA,
        pltpu.SemaphoreType.DMA,
        pltpu.SemaphoreType.DMA,
    ],
)
def _product_tuple(x_hbm, idx0_hbm, idx1_hbm, out_hbm,
                   i0s0, i1s0, i0s1, i1s1,
                   r0s0, r1s0, r0s1, r1s1,
                   ov0, ov1,
                   isem0, isem1, gsem0, gsem1, wsem0, wsem1):
    wid = lax.axis_index("s") * NC + lax.axis_index("c")
    base = wid * BPW

    islot = ((i0s0, i1s0), (i0s1, i1s1))
    rslot = ((r0s0, r1s0), (r0s1, r1s1))
    ov = (ov0, ov1)
    isem = (isem0, isem1)
    gsem = (gsem0, gsem1)
    wsem = (wsem0, wsem1)

    def off_of(c):
        return pl.multiple_of(base + c * C, 8)

    def issue_idx(c, b):
        off = off_of(c)
        pltpu.async_copy(idx0_hbm.at[pl.ds(off, C)], islot[b][0], isem[b])
        pltpu.async_copy(idx1_hbm.at[pl.ds(off, C)], islot[b][1], isem[b])

    def wait_idx(b):
        pltpu.make_async_copy(idx0_hbm.at[pl.ds(0, C)], islot[b][0], isem[b]).wait()
        pltpu.make_async_copy(idx1_hbm.at[pl.ds(0, C)], islot[b][1], isem[b]).wait()

    def issue_gather(b):
        pltpu.async_copy(x_hbm.at[islot[b][0]], rslot[b][0], gsem[b])
        pltpu.async_copy(x_hbm.at[islot[b][1]], rslot[b][1], gsem[b])

    def wait_gather(b):
        pltpu.make_async_copy(x_hbm.at[islot[b][0]], rslot[b][0], gsem[b]).wait()
        pltpu.make_async_copy(x_hbm.at[islot[b][1]], rslot[b][1], gsem[b]).wait()

    def compute(b):
        r0, r1, o = rslot[b][0], rslot[b][1], ov[b]

        def row_body(r, carry):
            for j in range(D // L):
                s = pl.ds(j * L, L)
                o[r, s] = r0[r, s] * r1[r, s]
            return carry

        lax.fori_loop(0, C, row_body, 0)

    def issue_wb(c, b):
        pltpu.async_copy(ov[b], out_hbm.at[pl.ds(off_of(c), C)], wsem[b])

    def wait_wb(b):
        pltpu.make_async_copy(ov[b], out_hbm.at[pl.ds(0, C)], wsem[b]).wait()

    issue_idx(0, 0)
    issue_idx(1, 1)
    wait_idx(0)
    issue_gather(0)

    wait_idx(1)
    wait_gather(0)
    issue_gather(1)
    issue_idx(2, 0)
    compute(0)
    issue_wb(0, 0)

    wait_idx(0)
    wait_gather(1)
    issue_gather(0)
    issue_idx(3, 1)
    compute(1)
    issue_wb(1, 1)

    def steady(i, carry):
        c0 = 2 + i * 2
        for b in range(2):
            c = c0 + b
            wait_idx(1 - b)
            wait_gather(b)
            issue_gather(1 - b)
            issue_idx(c + 2, b)
            wait_wb(b)
            compute(b)
            issue_wb(c, b)
        return carry

    lax.fori_loop(0, (N - 4) // 2, steady, 0)

    wait_idx(1)
    wait_gather(0)
    issue_gather(1)
    wait_wb(0)
    compute(0)
    issue_wb(N - 2, 0)

    wait_gather(1)
    wait_wb(1)
    compute(1)
    issue_wb(N - 1, 1)

    wait_wb(0)
    wait_wb(1)


def kernel(X, adj_t, tuples_coo):
    del adj_t
    idx0 = tuples_coo[0]
    idx1 = tuples_coo[1]
    return _product_tuple(X, idx0, idx1)


# speedup vs baseline: 13.8258x; 1.8910x over previous
import functools

import jax
import jax.numpy as jnp
from jax import lax
from jax.experimental import pallas as pl
from jax.experimental.pallas import tpu as pltpu
from jax.experimental.pallas import tpu_sc as plsc

V = 10000
D = 128
B = 320000
L = 16
NC = 2
NS = 16
NW = NC * NS
BPW = B // NW
C = 40
N = BPW // C

_mesh = plsc.VectorSubcoreMesh(core_axis_name="c", subcore_axis_name="s")


@functools.partial(
    pl.kernel,
    mesh=_mesh,
    out_type=jax.ShapeDtypeStruct((B, D), jnp.float32),
    scratch_types=[
        pltpu.VMEM((C,), jnp.int32),
        pltpu.VMEM((C,), jnp.int32),
        pltpu.VMEM((C,), jnp.int32),
        pltpu.VMEM((C,), jnp.int32),
        pltpu.VMEM((C, D), jnp.float32),
        pltpu.VMEM((C, D), jnp.float32),
        pltpu.VMEM((C, D), jnp.float32),
        pltpu.VMEM((C, D), jnp.float32),
        pltpu.VMEM((C, D), jnp.float32),
        pltpu.VMEM((C, D), jnp.float32),
        pltpu.VMEM_SHARED((V, D), jnp.float32),
        pltpu.SemaphoreType.DMA,
        pltpu.SemaphoreType.DMA,
        pltpu.SemaphoreType.DMA,
        pltpu.SemaphoreType.DMA,
        pltpu.SemaphoreType.DMA,
        pltpu.SemaphoreType.DMA,
    ],
)
def _product_tuple(x_hbm, idx0_hbm, idx1_hbm, out_hbm,
                   i0s0, i1s0, i0s1, i1s1,
                   r0s0, r1s0, r0s1, r1s1,
                   ov0, ov1, xs,
                   isem0, isem1, gsem0, gsem1, wsem0, wsem1):
    sid = lax.axis_index("s")
    wid = sid * NC + lax.axis_index("c")
    base = wid * BPW

    rows_per_sub = 624
    pltpu.sync_copy(x_hbm.at[pl.ds(sid * rows_per_sub, rows_per_sub)],
                    xs.at[pl.ds(sid * rows_per_sub, rows_per_sub)])

    @pl.when(sid == 0)
    def _stage_tail():
        tail = NS * rows_per_sub
        pltpu.sync_copy(x_hbm.at[pl.ds(tail, V - tail)],
                        xs.at[pl.ds(tail, V - tail)])

    plsc.subcore_barrier()

    islot = ((i0s0, i1s0), (i0s1, i1s1))
    rslot = ((r0s0, r1s0), (r0s1, r1s1))
    ov = (ov0, ov1)
    isem = (isem0, isem1)
    gsem = (gsem0, gsem1)
    wsem = (wsem0, wsem1)

    def off_of(c):
        return pl.multiple_of(base + c * C, 8)

    def issue_idx(c, b):
        off = off_of(c)
        pltpu.async_copy(idx0_hbm.at[pl.ds(off, C)], islot[b][0], isem[b])
        pltpu.async_copy(idx1_hbm.at[pl.ds(off, C)], islot[b][1], isem[b])

    def wait_idx(b):
        pltpu.make_async_copy(idx0_hbm.at[pl.ds(0, C)], islot[b][0], isem[b]).wait()
        pltpu.make_async_copy(idx1_hbm.at[pl.ds(0, C)], islot[b][1], isem[b]).wait()

    def issue_gather(b):
        pltpu.async_copy(xs.at[islot[b][0]], rslot[b][0], gsem[b])
        pltpu.async_copy(xs.at[islot[b][1]], rslot[b][1], gsem[b])

    def wait_gather(b):
        pltpu.make_async_copy(xs.at[islot[b][0]], rslot[b][0], gsem[b]).wait()
        pltpu.make_async_copy(xs.at[islot[b][1]], rslot[b][1], gsem[b]).wait()

    def compute(b):
        r0, r1, o = rslot[b][0], rslot[b][1], ov[b]

        def row_body(r, carry):
            for j in range(D // L):
                s = pl.ds(j * L, L)
                o[r, s] = r0[r, s] * r1[r, s]
            return carry

        lax.fori_loop(0, C, row_body, 0)

    def issue_wb(c, b):
        pltpu.async_copy(ov[b], out_hbm.at[pl.ds(off_of(c), C)], wsem[b])

    def wait_wb(b):
        pltpu.make_async_copy(ov[b], out_hbm.at[pl.ds(0, C)], wsem[b]).wait()

    issue_idx(0, 0)
    issue_idx(1, 1)
    wait_idx(0)
    issue_gather(0)

    wait_idx(1)
    wait_gather(0)
    issue_gather(1)
    issue_idx(2, 0)
    compute(0)
    issue_wb(0, 0)

    wait_idx(0)
    wait_gather(1)
    issue_gather(0)
    issue_idx(3, 1)
    compute(1)
    issue_wb(1, 1)

    def steady(i, carry):
        c0 = 2 + i * 2
        for b in range(2):
            c = c0 + b
            wait_idx(1 - b)
            wait_gather(b)
            issue_gather(1 - b)
            issue_idx(c + 2, b)
            wait_wb(b)
            compute(b)
            issue_wb(c, b)
        return carry

    lax.fori_loop(0, (N - 4) // 2, steady, 0)

    wait_idx(1)
    wait_gather(0)
    issue_gather(1)
    wait_wb(0)
    compute(0)
    issue_wb(N - 2, 0)

    wait_gather(1)
    wait_wb(1)
    compute(1)
    issue_wb(N - 1, 1)

    wait_wb(0)
    wait_wb(1)


def kernel(X, adj_t, tuples_coo):
    del adj_t
    idx0 = tuples_coo[0]
    idx1 = tuples_coo[1]
    return _product_tuple(X, idx0, idx1)
